# baseline (speedup 1.0000x reference)
import functools

import jax
import jax.numpy as jnp
from jax import lax
from jax.experimental import pallas as pl
from jax.experimental.pallas import tpu as pltpu
from jax.experimental.pallas import tpu_sc as plsc

NUM_NODES = 100000
H_DIM = 128

NC = 2
NS = 16
NW = NC * NS

CHUNK = 128
N_CHUNKS = 25
B_PAD = NW * N_CHUNKS * CHUNK

FULL_CHUNKS = NUM_NODES // CHUNK
TAIL_ROWS = NUM_NODES - FULL_CHUNKS * CHUNK

NBUF = 6
LEAD = 3


def _build():
    mesh = plsc.VectorSubcoreMesh(core_axis_name="c", subcore_axis_name="s")

    @functools.partial(
        pl.kernel,
        out_type=jax.ShapeDtypeStruct((NUM_NODES, H_DIM), jnp.float32),
        mesh=mesh,
        scratch_types=[
            pltpu.VMEM((N_CHUNKS, CHUNK), jnp.int32),
            pltpu.VMEM((NBUF, CHUNK, H_DIM), jnp.float32),
            pltpu.SemaphoreType.DMA((NBUF,)),
            pltpu.SemaphoreType.DMA((NBUF,)),
        ],
    )
    def gather_kernel(table_hbm, idx_hbm, out_hbm, idx_v, rows_v, gsem, osem):
        wid = lax.axis_index("s") * NC + lax.axis_index("c")
        c0 = wid * N_CHUNKS
        pltpu.sync_copy(idx_hbm.at[wid], idx_v)

        def fire_gather(j):
            @pl.when(c0 + j <= FULL_CHUNKS)
            def _():
                b = lax.rem(j, NBUF)
                pltpu.async_copy(
                    table_hbm.at[idx_v.at[j]], rows_v.at[b], gsem.at[b]
                )

        for j in range(LEAD):
            fire_gather(j)

        @pl.loop(0, N_CHUNKS)
        def _(j):
            b = lax.rem(j, NBUF)
            c = c0 + j

            @pl.when(j + LEAD < N_CHUNKS)
            def _():
                @pl.when(
                    jnp.logical_and(j - LEAD >= 0, c - LEAD < FULL_CHUNKS)
                )
                def _():
                    b2 = lax.rem(j + LEAD, NBUF)
                    pltpu.make_async_copy(
                        rows_v.at[b2],
                        out_hbm.at[pl.ds((c - LEAD) * CHUNK, CHUNK)],
                        osem.at[b2],
                    ).wait()

                fire_gather(j + LEAD)

            @pl.when(c <= FULL_CHUNKS)
            def _():
                pltpu.make_async_copy(
                    table_hbm.at[idx_v.at[j]], rows_v.at[b], gsem.at[b]
                ).wait()

            @pl.when(c < FULL_CHUNKS)
            def _():
                pltpu.async_copy(
                    rows_v.at[b], out_hbm.at[pl.ds(c * CHUNK, CHUNK)],
                    osem.at[b],
                )

            @pl.when(c == FULL_CHUNKS)
            def _():
                pltpu.sync_copy(
                    rows_v.at[b].at[pl.ds(0, TAIL_ROWS)],
                    out_hbm.at[pl.ds(FULL_CHUNKS * CHUNK, TAIL_ROWS)],
                )

        for j in range(N_CHUNKS - NBUF, N_CHUNKS):
            @pl.when(c0 + j < FULL_CHUNKS)
            def _(j=j):
                b = j % NBUF
                pltpu.make_async_copy(
                    rows_v.at[b],
                    out_hbm.at[pl.ds((c0 + j) * CHUNK, CHUNK)],
                    osem.at[b],
                ).wait()

    return gather_kernel


_GATHER = _build()


@jax.jit
def kernel(node_id, weight):
    idx = jnp.squeeze(node_id, axis=1)
    idx_pad = jnp.zeros((B_PAD,), jnp.int32).at[:NUM_NODES].set(idx)
    return _GATHER(weight, idx_pad.reshape(NW, N_CHUNKS, CHUNK))


# speedup vs baseline: 1.0047x; 1.0047x over previous
import functools

import jax
import jax.numpy as jnp
from jax import lax
from jax.experimental import pallas as pl
from jax.experimental.pallas import tpu as pltpu
from jax.experimental.pallas import tpu_sc as plsc

NUM_NODES = 100000
H_DIM = 128

NC = 2
NS = 16
NW = NC * NS

CHUNK = 128
N_CHUNKS = 25
PER_W = N_CHUNKS * CHUNK

FULL_CHUNKS = NUM_NODES // CHUNK
TAIL_ROWS = NUM_NODES - FULL_CHUNKS * CHUNK
LAST_W = FULL_CHUNKS // N_CHUNKS
LAST_W_VALID = NUM_NODES - LAST_W * PER_W

NBUF = 6
LEAD = 3

LANES = 16


def _build():
    mesh = plsc.VectorSubcoreMesh(core_axis_name="c", subcore_axis_name="s")

    @functools.partial(
        pl.kernel,
        out_type=jax.ShapeDtypeStruct((NUM_NODES, H_DIM), jnp.float32),
        mesh=mesh,
        scratch_types=[
            pltpu.VMEM((PER_W,), jnp.int32),
            pltpu.VMEM((NBUF, CHUNK, H_DIM), jnp.float32),
            pltpu.SemaphoreType.DMA((NBUF,)),
            pltpu.SemaphoreType.DMA((NBUF,)),
        ],
    )
    def gather_kernel(table_hbm, idx_hbm, out_hbm, idx_v, rows_v, gsem, osem):
        wid = lax.axis_index("s") * NC + lax.axis_index("c")
        c0 = wid * N_CHUNKS

        @pl.when(wid < LAST_W)
        def _():
            pltpu.sync_copy(idx_hbm.at[pl.ds(wid * PER_W, PER_W)], idx_v)

        @pl.when(wid == LAST_W)
        def _():
            pltpu.sync_copy(
                idx_hbm.at[pl.ds(LAST_W * PER_W, LAST_W_VALID)],
                idx_v.at[pl.ds(0, LAST_W_VALID)],
            )
            for k in range((CHUNK - TAIL_ROWS) // LANES):
                idx_v[pl.ds(LAST_W_VALID + k * LANES, LANES)] = jnp.zeros(
                    (LANES,), jnp.int32
                )

        def fire_gather(j):
            @pl.when(c0 + j <= FULL_CHUNKS)
            def _():
                b = lax.rem(j, NBUF)
                pltpu.async_copy(
                    table_hbm.at[idx_v.at[pl.ds(j * CHUNK, CHUNK)]],
                    rows_v.at[b],
                    gsem.at[b],
                )

        for j in range(LEAD):
            fire_gather(j)

        @pl.loop(0, N_CHUNKS)
        def _(j):
            b = lax.rem(j, NBUF)
            c = c0 + j

            @pl.when(j + LEAD < N_CHUNKS)
            def _():
                @pl.when(
                    jnp.logical_and(j - LEAD >= 0, c - LEAD < FULL_CHUNKS)
                )
                def _():
                    b2 = lax.rem(j + LEAD, NBUF)
                    pltpu.make_async_copy(
                        rows_v.at[b2],
                        out_hbm.at[pl.ds((c - LEAD) * CHUNK, CHUNK)],
                        osem.at[b2],
                    ).wait()

                fire_gather(j + LEAD)

            @pl.when(c <= FULL_CHUNKS)
            def _():
                pltpu.make_async_copy(
                    table_hbm.at[idx_v.at[pl.ds(j * CHUNK, CHUNK)]],
                    rows_v.at[b],
                    gsem.at[b],
                ).wait()

            @pl.when(c < FULL_CHUNKS)
            def _():
                pltpu.async_copy(
                    rows_v.at[b], out_hbm.at[pl.ds(c * CHUNK, CHUNK)],
                    osem.at[b],
                )

            @pl.when(c == FULL_CHUNKS)
            def _():
                pltpu.sync_copy(
                    rows_v.at[b].at[pl.ds(0, TAIL_ROWS)],
                    out_hbm.at[pl.ds(FULL_CHUNKS * CHUNK, TAIL_ROWS)],
                )

        for j in range(N_CHUNKS - NBUF, N_CHUNKS):
            @pl.when(c0 + j < FULL_CHUNKS)
            def _(j=j):
                b = j % NBUF
                pltpu.make_async_copy(
                    rows_v.at[b],
                    out_hbm.at[pl.ds((c0 + j) * CHUNK, CHUNK)],
                    osem.at[b],
                ).wait()

    return gather_kernel


_GATHER = _build()


@jax.jit
def kernel(node_id, weight):
    return _GATHER(weight, node_id.reshape(NUM_NODES))


# speedup vs baseline: 1.0504x; 1.0455x over previous
import functools

import jax
import jax.numpy as jnp
from jax import lax
from jax.experimental import pallas as pl
from jax.experimental.pallas import tpu as pltpu
from jax.experimental.pallas import tpu_sc as plsc

NUM_NODES = 100000
H_DIM = 128

NC = 2
NS = 16
NW = NC * NS

CHUNK = 128
K0 = 27
K1 = 23

FULL_CHUNKS = NUM_NODES // CHUNK
TAIL_ROWS = NUM_NODES - FULL_CHUNKS * CHUNK

LAST_BASE = (NS * K0 + (NS - 1) * K1) * CHUNK
LAST_VALID = NUM_NODES - LAST_BASE
LAST_JUNK = 96

NBUF = 6
LEAD = 3

LANES = 16


def _build():
    mesh = plsc.VectorSubcoreMesh(core_axis_name="c", subcore_axis_name="s")

    @functools.partial(
        pl.kernel,
        out_type=jax.ShapeDtypeStruct((NUM_NODES, H_DIM), jnp.float32),
        mesh=mesh,
        scratch_types=[
            pltpu.VMEM((K0 * CHUNK,), jnp.int32),
            pltpu.VMEM((NBUF, CHUNK, H_DIM), jnp.float32),
            pltpu.SemaphoreType.DMA((NBUF,)),
            pltpu.SemaphoreType.DMA((NBUF,)),
        ],
    )
    def gather_kernel(table_hbm, idx_hbm, out_hbm, idx_v, rows_v, gsem, osem):
        cix = lax.axis_index("c")
        six = lax.axis_index("s")
        on_c0 = cix == 0
        k = jnp.where(on_c0, K0, K1)
        c0 = jnp.where(on_c0, six * K0, NS * K0 + six * K1)
        last = jnp.logical_and(cix == 1, six == NS - 1)

        @pl.when(on_c0)
        def _():
            pltpu.sync_copy(idx_hbm.at[pl.ds(c0 * CHUNK, K0 * CHUNK)], idx_v)

        @pl.when(jnp.logical_and(cix == 1, jnp.logical_not(last)))
        def _():
            pltpu.sync_copy(
                idx_hbm.at[pl.ds(c0 * CHUNK, K1 * CHUNK)],
                idx_v.at[pl.ds(0, K1 * CHUNK)],
            )

        @pl.when(last)
        def _():
            pltpu.sync_copy(
                idx_hbm.at[pl.ds(LAST_BASE, LAST_VALID)],
                idx_v.at[pl.ds(0, LAST_VALID)],
            )
            for kk in range(LAST_JUNK // LANES):
                idx_v[pl.ds(LAST_VALID + kk * LANES, LANES)] = jnp.zeros(
                    (LANES,), jnp.int32
                )

        def fire_gather(j):
            @pl.when(c0 + j <= FULL_CHUNKS)
            def _():
                b = lax.rem(j, NBUF)
                pltpu.async_copy(
                    table_hbm.at[idx_v.at[pl.ds(j * CHUNK, CHUNK)]],
                    rows_v.at[b],
                    gsem.at[b],
                )

        for j in range(LEAD):
            fire_gather(j)

        @pl.loop(0, k)
        def _(j):
            b = lax.rem(j, NBUF)
            c = c0 + j

            @pl.when(j + LEAD < k)
            def _():
                @pl.when(
                    jnp.logical_and(j - LEAD >= 0, c - LEAD < FULL_CHUNKS)
                )
                def _():
                    b2 = lax.rem(j + LEAD, NBUF)
                    pltpu.make_async_copy(
                        rows_v.at[b2],
                        out_hbm.at[pl.ds((c - LEAD) * CHUNK, CHUNK)],
                        osem.at[b2],
                    ).wait()

                fire_gather(j + LEAD)

            @pl.when(c <= FULL_CHUNKS)
            def _():
                pltpu.make_async_copy(
                    table_hbm.at[idx_v.at[pl.ds(j * CHUNK, CHUNK)]],
                    rows_v.at[b],
                    gsem.at[b],
                ).wait()

            @pl.when(c < FULL_CHUNKS)
            def _():
                pltpu.async_copy(
                    rows_v.at[b], out_hbm.at[pl.ds(c * CHUNK, CHUNK)],
                    osem.at[b],
                )

            @pl.when(c == FULL_CHUNKS)
            def _():
                pltpu.sync_copy(
                    rows_v.at[b].at[pl.ds(0, TAIL_ROWS)],
                    out_hbm.at[pl.ds(FULL_CHUNKS * CHUNK, TAIL_ROWS)],
                )

        def drain(j):
            @pl.when(c0 + j < FULL_CHUNKS)
            def _():
                b = j % NBUF
                pltpu.make_async_copy(
                    rows_v.at[b],
                    out_hbm.at[pl.ds((c0 + j) * CHUNK, CHUNK)],
                    osem.at[b],
                ).wait()

        @pl.when(on_c0)
        def _():
            for j in range(K0 - NBUF, K0):
                drain(j)

        @pl.when(jnp.logical_not(on_c0))
        def _():
            for j in range(K1 - NBUF, K1):
                drain(j)

    return gather_kernel


_GATHER = _build()


@jax.jit
def kernel(node_id, weight):
    return _GATHER(weight, node_id.reshape(NUM_NODES))


# speedup vs baseline: 1.1095x; 1.0563x over previous
import functools

import jax
import jax.numpy as jnp
from jax import lax
from jax.experimental import pallas as pl
from jax.experimental.pallas import tpu as pltpu
from jax.experimental.pallas import tpu_sc as plsc

NUM_NODES = 100000
H_DIM = 128

NC = 2
NS = 16
NW = NC * NS

CHUNK = 128
K0 = 28
K1 = 22

FULL_CHUNKS = NUM_NODES // CHUNK
TAIL_ROWS = NUM_NODES - FULL_CHUNKS * CHUNK

LAST_BASE = (NS * K0 + (NS - 1) * K1) * CHUNK
LAST_VALID = NUM_NODES - LAST_BASE
LAST_JUNK = 96

NBUF = 6
LEAD = 3

LANES = 16


def _build():
    mesh = plsc.VectorSubcoreMesh(core_axis_name="c", subcore_axis_name="s")

    @functools.partial(
        pl.kernel,
        out_type=jax.ShapeDtypeStruct((NUM_NODES, H_DIM), jnp.float32),
        mesh=mesh,
        scratch_types=[
            pltpu.VMEM((K0 * CHUNK,), jnp.int32),
            pltpu.VMEM((NBUF, CHUNK, H_DIM), jnp.float32),
            pltpu.SemaphoreType.DMA((NBUF,)),
            pltpu.SemaphoreType.DMA((NBUF,)),
        ],
    )
    def gather_kernel(table_hbm, idx_hbm, out_hbm, idx_v, rows_v, gsem, osem):
        cix = lax.axis_index("c")
        six = lax.axis_index("s")
        on_c0 = cix == 0
        k = jnp.where(on_c0, K0, K1)
        c0 = jnp.where(on_c0, six * K0, NS * K0 + six * K1)
        last = jnp.logical_and(cix == 1, six == NS - 1)

        @pl.when(on_c0)
        def _():
            pltpu.sync_copy(idx_hbm.at[pl.ds(c0 * CHUNK, K0 * CHUNK)], idx_v)

        @pl.when(jnp.logical_and(cix == 1, jnp.logical_not(last)))
        def _():
            pltpu.sync_copy(
                idx_hbm.at[pl.ds(c0 * CHUNK, K1 * CHUNK)],
                idx_v.at[pl.ds(0, K1 * CHUNK)],
            )

        @pl.when(last)
        def _():
            pltpu.sync_copy(
                idx_hbm.at[pl.ds(LAST_BASE, LAST_VALID)],
                idx_v.at[pl.ds(0, LAST_VALID)],
            )
            for kk in range(LAST_JUNK // LANES):
                idx_v[pl.ds(LAST_VALID + kk * LANES, LANES)] = jnp.zeros(
                    (LANES,), jnp.int32
                )

        def fire_gather(j):
            @pl.when(c0 + j <= FULL_CHUNKS)
            def _():
                b = lax.rem(j, NBUF)
                pltpu.async_copy(
                    table_hbm.at[idx_v.at[pl.ds(j * CHUNK, CHUNK)]],
                    rows_v.at[b],
                    gsem.at[b],
                )

        for j in range(LEAD):
            fire_gather(j)

        @pl.loop(0, k)
        def _(j):
            b = lax.rem(j, NBUF)
            c = c0 + j

            @pl.when(j + LEAD < k)
            def _():
                @pl.when(
                    jnp.logical_and(j - LEAD >= 0, c - LEAD < FULL_CHUNKS)
                )
                def _():
                    b2 = lax.rem(j + LEAD, NBUF)
                    pltpu.make_async_copy(
                        rows_v.at[b2],
                        out_hbm.at[pl.ds((c - LEAD) * CHUNK, CHUNK)],
                        osem.at[b2],
                    ).wait()

                fire_gather(j + LEAD)

            @pl.when(c <= FULL_CHUNKS)
            def _():
                pltpu.make_async_copy(
                    table_hbm.at[idx_v.at[pl.ds(j * CHUNK, CHUNK)]],
                    rows_v.at[b],
                    gsem.at[b],
                ).wait()

            @pl.when(c < FULL_CHUNKS)
            def _():
                pltpu.async_copy(
                    rows_v.at[b], out_hbm.at[pl.ds(c * CHUNK, CHUNK)],
                    osem.at[b],
                )

            @pl.when(c == FULL_CHUNKS)
            def _():
                pltpu.sync_copy(
                    rows_v.at[b].at[pl.ds(0, TAIL_ROWS)],
                    out_hbm.at[pl.ds(FULL_CHUNKS * CHUNK, TAIL_ROWS)],
                )

        def drain(j):
            @pl.when(c0 + j < FULL_CHUNKS)
            def _():
                b = j % NBUF
                pltpu.make_async_copy(
                    rows_v.at[b],
                    out_hbm.at[pl.ds((c0 + j) * CHUNK, CHUNK)],
                    osem.at[b],
                ).wait()

        @pl.when(on_c0)
        def _():
            for j in range(K0 - NBUF, K0):
                drain(j)

        @pl.when(jnp.logical_not(on_c0))
        def _():
            for j in range(K1 - NBUF, K1):
                drain(j)

    return gather_kernel


_GATHER = _build()


@jax.jit
def kernel(node_id, weight):
    return _GATHER(weight, node_id.reshape(NUM_NODES))


# speedup vs baseline: 1.1317x; 1.0199x over previous
import functools

import jax
import jax.numpy as jnp
from jax import lax
from jax.experimental import pallas as pl
from jax.experimental.pallas import tpu as pltpu
from jax.experimental.pallas import tpu_sc as plsc

NUM_NODES = 100000
H_DIM = 128

NC = 2
NS = 16
NW = NC * NS

CHUNK = 128
K0 = 29
K1 = 21

FULL_CHUNKS = NUM_NODES // CHUNK
TAIL_ROWS = NUM_NODES - FULL_CHUNKS * CHUNK

LAST_BASE = (NS * K0 + (NS - 1) * K1) * CHUNK
LAST_VALID = NUM_NODES - LAST_BASE
LAST_JUNK = 96

NBUF = 6
LEAD = 3

LANES = 16


def _build():
    mesh = plsc.VectorSubcoreMesh(core_axis_name="c", subcore_axis_name="s")

    @functools.partial(
        pl.kernel,
        out_type=jax.ShapeDtypeStruct((NUM_NODES, H_DIM), jnp.float32),
        mesh=mesh,
        scratch_types=[
            pltpu.VMEM((K0 * CHUNK,), jnp.int32),
            pltpu.VMEM((NBUF, CHUNK, H_DIM), jnp.float32),
            pltpu.SemaphoreType.DMA((NBUF,)),
            pltpu.SemaphoreType.DMA((NBUF,)),
        ],
    )
    def gather_kernel(table_hbm, idx_hbm, out_hbm, idx_v, rows_v, gsem, osem):
        cix = lax.axis_index("c")
        six = lax.axis_index("s")
        on_c0 = cix == 0
        k = jnp.where(on_c0, K0, K1)
        c0 = jnp.where(on_c0, six * K0, NS * K0 + six * K1)
        last = jnp.logical_and(cix == 1, six == NS - 1)

        @pl.when(on_c0)
        def _():
            pltpu.sync_copy(idx_hbm.at[pl.ds(c0 * CHUNK, K0 * CHUNK)], idx_v)

        @pl.when(jnp.logical_and(cix == 1, jnp.logical_not(last)))
        def _():
            pltpu.sync_copy(
                idx_hbm.at[pl.ds(c0 * CHUNK, K1 * CHUNK)],
                idx_v.at[pl.ds(0, K1 * CHUNK)],
            )

        @pl.when(last)
        def _():
            pltpu.sync_copy(
                idx_hbm.at[pl.ds(LAST_BASE, LAST_VALID)],
                idx_v.at[pl.ds(0, LAST_VALID)],
            )
            for kk in range(LAST_JUNK // LANES):
                idx_v[pl.ds(LAST_VALID + kk * LANES, LANES)] = jnp.zeros(
                    (LANES,), jnp.int32
                )

        def fire_gather(j):
            @pl.when(c0 + j <= FULL_CHUNKS)
            def _():
                b = lax.rem(j, NBUF)
                pltpu.async_copy(
                    table_hbm.at[idx_v.at[pl.ds(j * CHUNK, CHUNK)]],
                    rows_v.at[b],
                    gsem.at[b],
                )

        for j in range(LEAD):
            fire_gather(j)

        @pl.loop(0, k)
        def _(j):
            b = lax.rem(j, NBUF)
            c = c0 + j

            @pl.when(j + LEAD < k)
            def _():
                @pl.when(
                    jnp.logical_and(j - LEAD >= 0, c - LEAD < FULL_CHUNKS)
                )
                def _():
                    b2 = lax.rem(j + LEAD, NBUF)
                    pltpu.make_async_copy(
                        rows_v.at[b2],
                        out_hbm.at[pl.ds((c - LEAD) * CHUNK, CHUNK)],
                        osem.at[b2],
                    ).wait()

                fire_gather(j + LEAD)

            @pl.when(c <= FULL_CHUNKS)
            def _():
                pltpu.make_async_copy(
                    table_hbm.at[idx_v.at[pl.ds(j * CHUNK, CHUNK)]],
                    rows_v.at[b],
                    gsem.at[b],
                ).wait()

            @pl.when(c < FULL_CHUNKS)
            def _():
                pltpu.async_copy(
                    rows_v.at[b], out_hbm.at[pl.ds(c * CHUNK, CHUNK)],
                    osem.at[b],
                )

            @pl.when(c == FULL_CHUNKS)
            def _():
                pltpu.sync_copy(
                    rows_v.at[b].at[pl.ds(0, TAIL_ROWS)],
                    out_hbm.at[pl.ds(FULL_CHUNKS * CHUNK, TAIL_ROWS)],
                )

        def drain(j):
            @pl.when(c0 + j < FULL_CHUNKS)
            def _():
                b = j % NBUF
                pltpu.make_async_copy(
                    rows_v.at[b],
                    out_hbm.at[pl.ds((c0 + j) * CHUNK, CHUNK)],
                    osem.at[b],
                ).wait()

        @pl.when(on_c0)
        def _():
            for j in range(K0 - NBUF, K0):
                drain(j)

        @pl.when(jnp.logical_not(on_c0))
        def _():
            for j in range(K1 - NBUF, K1):
                drain(j)

    return gather_kernel


_GATHER = _build()


@jax.jit
def kernel(node_id, weight):
    return _GATHER(weight, node_id.reshape(NUM_NODES))
